# Initial kernel scaffold; baseline (speedup 1.0000x reference)
import jax
import jax.numpy as jnp
from jax.experimental import pallas as pl


def kernel(country, os, browser, device_type, country_table, os_table, browser_table, device_type_table, W, b):
    raise NotImplementedError("write your pallas kernel here")


# speedup vs baseline: 4.4699x; 4.4699x over previous
import functools

import jax
import jax.numpy as jnp
import numpy as np
from jax import lax
from jax.experimental import pallas as pl
from jax.experimental.pallas import tpu as pltpu
from jax.experimental.pallas import tpu_sc as plsc

B = 16384
D = 256
NC = 2
NS = 16
NW = NC * NS
RPW = B // NW
CH = 128
NCHUNK = RPW // CH

_i500 = np.arange(500)
_RO = (_i500[:, None] // 50 == np.arange(10)[None, :]).astype(np.float32)
_RB = ((_i500[:, None] // 5) % 10 == np.arange(10)[None, :]).astype(np.float32)
_RD = (_i500[:, None] % 5 == np.arange(5)[None, :]).astype(np.float32)


def _tables_body(ct_ref, ot_ref, bt_ref, dt_ref, w_ref, b_ref,
                 ro_ref, rb_ref, rd_ref, pc_ref, ps_ref):
    w = w_ref[...]
    dn = (((1,), (1,)), ((), ()))
    pc_ref[...] = lax.dot_general(ct_ref[...], w[:, 0:32], dn,
                                  preferred_element_type=jnp.float32)
    po = lax.dot_general(ot_ref[...], w[:, 32:48], dn,
                         preferred_element_type=jnp.float32)
    pb = lax.dot_general(bt_ref[...], w[:, 48:64], dn,
                         preferred_element_type=jnp.float32)
    pd = lax.dot_general(dt_ref[...], w[:, 64:72], dn,
                         preferred_element_type=jnp.float32)
    dn2 = (((1,), (0,)), ((), ()))
    ps = (lax.dot_general(ro_ref[...], po, dn2, preferred_element_type=jnp.float32)
          + lax.dot_general(rb_ref[...], pb, dn2, preferred_element_type=jnp.float32)
          + lax.dot_general(rd_ref[...], pd, dn2, preferred_element_type=jnp.float32)
          + b_ref[...])
    ps_ref[...] = ps


def _build_tables(ct, ot, bt, dt, w, b):
    return pl.pallas_call(
        _tables_body,
        out_shape=(
            jax.ShapeDtypeStruct((250, D), jnp.float32),
            jax.ShapeDtypeStruct((500, D), jnp.float32),
        ),
    )(ct, ot, bt, dt, w, b.reshape(1, D), jnp.asarray(_RO), jnp.asarray(_RB),
      jnp.asarray(_RD))


_SC_ENCODE_CACHE = []


def _get_sc_encode():
    if _SC_ENCODE_CACHE:
        return _SC_ENCODE_CACHE[0]
    mesh = plsc.VectorSubcoreMesh(core_axis_name="c", subcore_axis_name="s",
                                  num_cores=NC, num_subcores=NS)

    @functools.partial(
        pl.kernel,
        out_type=jax.ShapeDtypeStruct((B, D), jnp.float32),
        mesh=mesh,
        scratch_types=[
            pltpu.VMEM((NCHUNK, CH), jnp.int32),
            pltpu.VMEM((NCHUNK, CH), jnp.int32),
            pltpu.VMEM((RPW,), jnp.int32),
            pltpu.VMEM((RPW,), jnp.int32),
            pltpu.VMEM((RPW,), jnp.int32),
            pltpu.VMEM((CH, D), jnp.float32),
            pltpu.VMEM((CH, D), jnp.float32),
            pltpu.SemaphoreType.DMA,
            pltpu.SemaphoreType.DMA,
        ],
    )
    def _sc_encode(pc_hbm, ps_hbm, country_hbm, os_hbm, br_hbm, dv_hbm, out_hbm,
                   idc, ids, sto, stb, std, buf_a, buf_b, sem_a, sem_b):
        wid = lax.axis_index("s") * NC + lax.axis_index("c")
        base = wid * RPW
        for ch in range(NCHUNK):
            pltpu.sync_copy(country_hbm.at[pl.ds(base + ch * CH, CH)], idc.at[ch])
        pltpu.sync_copy(os_hbm.at[pl.ds(base, RPW)], sto)
        pltpu.sync_copy(br_hbm.at[pl.ds(base, RPW)], stb)
        pltpu.sync_copy(dv_hbm.at[pl.ds(base, RPW)], std)
        for i in range(RPW // 16):
            ch, j = i // (CH // 16), i % (CH // 16)
            sl = pl.ds(i * 16, 16)
            ids[ch, pl.ds(j * 16, 16)] = sto[sl] * 50 + stb[sl] * 5 + std[sl]
        for ch in range(NCHUNK):
            cp_a = pltpu.async_copy(pc_hbm.at[idc.at[ch]], buf_a, sem_a)
            cp_b = pltpu.async_copy(ps_hbm.at[ids.at[ch]], buf_b, sem_b)
            cp_a.wait()
            cp_b.wait()

            def _add_row(r, _):
                for j in range(D // 16):
                    sl = pl.ds(j * 16, 16)
                    buf_a[r, sl] = buf_a[r, sl] + buf_b[r, sl]
                return 0

            lax.fori_loop(0, CH, _add_row, 0)
            pltpu.sync_copy(buf_a, out_hbm.at[pl.ds(base + ch * CH, CH)])

    _SC_ENCODE_CACHE.append(_sc_encode)
    return _sc_encode


def kernel(country, os, browser, device_type, country_table, os_table,
           browser_table, device_type_table, W, b):
    pc, ps = _build_tables(country_table, os_table, browser_table,
                           device_type_table, W, b)
    return _get_sc_encode()(
        pc, ps,
        country.astype(jnp.int32), os.astype(jnp.int32),
        browser.astype(jnp.int32), device_type.astype(jnp.int32))


# speedup vs baseline: 4.8708x; 1.0897x over previous
import functools

import jax
import jax.numpy as jnp
import numpy as np
from jax import lax
from jax.experimental import pallas as pl
from jax.experimental.pallas import tpu as pltpu
from jax.experimental.pallas import tpu_sc as plsc

B = 16384
D = 256
NC = 2
NS = 16
NW = NC * NS
RPW = B // NW
CH = 64
NCHUNK = RPW // CH

_i500 = np.arange(500)
_RO = (_i500[:, None] // 50 == np.arange(10)[None, :]).astype(np.float32)
_RB = ((_i500[:, None] // 5) % 10 == np.arange(10)[None, :]).astype(np.float32)
_RD = (_i500[:, None] % 5 == np.arange(5)[None, :]).astype(np.float32)


def _tables_body(ct_ref, ot_ref, bt_ref, dt_ref, w_ref, b_ref,
                 ro_ref, rb_ref, rd_ref, pc_ref, ps_ref):
    w = w_ref[...]
    dn = (((1,), (1,)), ((), ()))
    pc_ref[...] = lax.dot_general(ct_ref[...], w[:, 0:32], dn,
                                  preferred_element_type=jnp.float32)
    po = lax.dot_general(ot_ref[...], w[:, 32:48], dn,
                         preferred_element_type=jnp.float32)
    pb = lax.dot_general(bt_ref[...], w[:, 48:64], dn,
                         preferred_element_type=jnp.float32)
    pd = lax.dot_general(dt_ref[...], w[:, 64:72], dn,
                         preferred_element_type=jnp.float32)
    dn2 = (((1,), (0,)), ((), ()))
    ps = (lax.dot_general(ro_ref[...], po, dn2, preferred_element_type=jnp.float32)
          + lax.dot_general(rb_ref[...], pb, dn2, preferred_element_type=jnp.float32)
          + lax.dot_general(rd_ref[...], pd, dn2, preferred_element_type=jnp.float32)
          + b_ref[...])
    ps_ref[...] = ps


def _build_tables(ct, ot, bt, dt, w, b):
    return pl.pallas_call(
        _tables_body,
        out_shape=(
            jax.ShapeDtypeStruct((250, D), jnp.float32),
            jax.ShapeDtypeStruct((500, D), jnp.float32),
        ),
    )(ct, ot, bt, dt, w, b.reshape(1, D), jnp.asarray(_RO), jnp.asarray(_RB),
      jnp.asarray(_RD))


_SC_ENCODE_CACHE = []


def _get_sc_encode():
    if _SC_ENCODE_CACHE:
        return _SC_ENCODE_CACHE[0]
    mesh = plsc.VectorSubcoreMesh(core_axis_name="c", subcore_axis_name="s",
                                  num_cores=NC, num_subcores=NS)

    @functools.partial(
        pl.kernel,
        out_type=jax.ShapeDtypeStruct((B, D), jnp.float32),
        mesh=mesh,
        scratch_types=[
            pltpu.VMEM((RPW,), jnp.int32),
            pltpu.VMEM((RPW,), jnp.int32),
            pltpu.VMEM((RPW,), jnp.int32),
            pltpu.VMEM((RPW,), jnp.int32),
            pltpu.VMEM((RPW,), jnp.int32),
            pltpu.VMEM((CH, D), jnp.float32),
            pltpu.VMEM((CH, D), jnp.float32),
            pltpu.VMEM((CH, D), jnp.float32),
            pltpu.VMEM((CH, D), jnp.float32),
            pltpu.SemaphoreType.DMA,
            pltpu.SemaphoreType.DMA,
            pltpu.SemaphoreType.DMA,
            pltpu.SemaphoreType.DMA,
            pltpu.SemaphoreType.DMA,
            pltpu.SemaphoreType.DMA,
        ],
    )
    def _sc_encode(pc_hbm, ps_hbm, country_hbm, os_hbm, br_hbm, dv_hbm, out_hbm,
                   idc, ids, sto, stb, std, ba0, ba1, bb0, bb1,
                   sa0, sa1, sb0, sb1, so0, so1):
        wid = lax.axis_index("s") * NC + lax.axis_index("c")
        base = wid * RPW
        pltpu.sync_copy(country_hbm.at[pl.ds(base, RPW)], idc)
        pltpu.sync_copy(os_hbm.at[pl.ds(base, RPW)], sto)
        pltpu.sync_copy(br_hbm.at[pl.ds(base, RPW)], stb)
        pltpu.sync_copy(dv_hbm.at[pl.ds(base, RPW)], std)
        for i in range(RPW // 16):
            sl = pl.ds(i * 16, 16)
            ids[sl] = sto[sl] * 50 + stb[sl] * 5 + std[sl]
        buf_a, buf_b = (ba0, ba1), (bb0, bb1)
        sem_a, sem_b, sem_o = (sa0, sa1), (sb0, sb1), (so0, so1)

        def fire(ch):
            s = ch % 2
            isl = pl.ds(ch * CH, CH)
            return (pltpu.async_copy(pc_hbm.at[idc.at[isl]], buf_a[s], sem_a[s]),
                    pltpu.async_copy(ps_hbm.at[ids.at[isl]], buf_b[s], sem_b[s]))

        gathers = {0: fire(0)}
        stores = {}
        for ch in range(NCHUNK):
            s = ch % 2
            if ch >= 1:
                stores[ch - 1].wait()
            if ch + 1 < NCHUNK:
                gathers[ch + 1] = fire(ch + 1)
            ga, gb = gathers[ch]
            ga.wait()
            gb.wait()
            a, bb = buf_a[s], buf_b[s]

            def _add_row(r, _, a=a, bb=bb):
                for j in range(D // 16):
                    sl2 = pl.ds(j * 16, 16)
                    plsc.addupdate(a.at[r, sl2], bb[r, sl2])
                return 0

            lax.fori_loop(0, CH, _add_row, 0)
            stores[ch] = pltpu.async_copy(
                a, out_hbm.at[pl.ds(base + ch * CH, CH)], sem_o[s])
        stores[NCHUNK - 1].wait()

    _SC_ENCODE_CACHE.append(_sc_encode)
    return _sc_encode


def kernel(country, os, browser, device_type, country_table, os_table,
           browser_table, device_type_table, W, b):
    pc, ps = _build_tables(country_table, os_table, browser_table,
                           device_type_table, W, b)
    return _get_sc_encode()(
        pc, ps,
        country.astype(jnp.int32), os.astype(jnp.int32),
        browser.astype(jnp.int32), device_type.astype(jnp.int32))


# speedup vs baseline: 5.1122x; 1.0496x over previous
import functools

import jax
import jax.numpy as jnp
import numpy as np
from jax import lax
from jax.experimental import pallas as pl
from jax.experimental.pallas import tpu as pltpu
from jax.experimental.pallas import tpu_sc as plsc

B = 16384
D = 256
NC = 2
NS = 16
NW = NC * NS
RPW = B // NW
CH = 64
NCHUNK = RPW // CH

_i500 = np.arange(500)
_RO = (_i500[:, None] // 50 == np.arange(10)[None, :]).astype(np.float32)
_RB = ((_i500[:, None] // 5) % 10 == np.arange(10)[None, :]).astype(np.float32)
_RD = (_i500[:, None] % 5 == np.arange(5)[None, :]).astype(np.float32)


def _tables_body(ct_ref, ot_ref, bt_ref, dt_ref, w_ref, b_ref,
                 ro_ref, rb_ref, rd_ref, pc_ref, ps_ref):
    w = w_ref[...]
    dn = (((1,), (1,)), ((), ()))
    pc_ref[...] = lax.dot_general(ct_ref[...], w[:, 0:32], dn,
                                  preferred_element_type=jnp.float32)
    po = lax.dot_general(ot_ref[...], w[:, 32:48], dn,
                         preferred_element_type=jnp.float32)
    pb = lax.dot_general(bt_ref[...], w[:, 48:64], dn,
                         preferred_element_type=jnp.float32)
    pd = lax.dot_general(dt_ref[...], w[:, 64:72], dn,
                         preferred_element_type=jnp.float32)
    dn2 = (((1,), (0,)), ((), ()))
    ps = (lax.dot_general(ro_ref[...], po, dn2, preferred_element_type=jnp.float32)
          + lax.dot_general(rb_ref[...], pb, dn2, preferred_element_type=jnp.float32)
          + lax.dot_general(rd_ref[...], pd, dn2, preferred_element_type=jnp.float32)
          + b_ref[...])
    ps_ref[...] = ps


def _build_tables(ct, ot, bt, dt, w, b):
    return pl.pallas_call(
        _tables_body,
        out_shape=(
            jax.ShapeDtypeStruct((250, D), jnp.float32),
            jax.ShapeDtypeStruct((500, D), jnp.float32),
        ),
    )(ct, ot, bt, dt, w, b.reshape(1, D), jnp.asarray(_RO), jnp.asarray(_RB),
      jnp.asarray(_RD))


_SC_ENCODE_CACHE = []


def _get_sc_encode():
    if _SC_ENCODE_CACHE:
        return _SC_ENCODE_CACHE[0]
    mesh = plsc.VectorSubcoreMesh(core_axis_name="c", subcore_axis_name="s",
                                  num_cores=NC, num_subcores=NS)

    @functools.partial(
        pl.kernel,
        out_type=jax.ShapeDtypeStruct((B, D), jnp.float32),
        mesh=mesh,
        scratch_types=[
            pltpu.VMEM((RPW,), jnp.int32),
            pltpu.VMEM((RPW,), jnp.int32),
            pltpu.VMEM((RPW,), jnp.int32),
            pltpu.VMEM((RPW,), jnp.int32),
            pltpu.VMEM((RPW,), jnp.int32),
            pltpu.VMEM((CH, D), jnp.float32),
            pltpu.VMEM((CH, D), jnp.float32),
            pltpu.VMEM((CH, D), jnp.float32),
            pltpu.VMEM((CH, D), jnp.float32),
            pltpu.VMEM((CH, D), jnp.float32),
            pltpu.VMEM((CH, D), jnp.float32),
            pltpu.SemaphoreType.DMA,
            pltpu.SemaphoreType.DMA,
            pltpu.SemaphoreType.DMA,
            pltpu.SemaphoreType.DMA,
            pltpu.SemaphoreType.DMA,
            pltpu.SemaphoreType.DMA,
            pltpu.SemaphoreType.DMA,
            pltpu.SemaphoreType.DMA,
            pltpu.SemaphoreType.DMA,
            pltpu.SemaphoreType.DMA,
        ],
    )
    def _sc_encode(pc_hbm, ps_hbm, country_hbm, os_hbm, br_hbm, dv_hbm, out_hbm,
                   idc, ids, sto, stb, std, ba0, ba1, ba2, bb0, bb1, bb2,
                   sa0, sa1, sa2, sb0, sb1, sb2, so0, so1, so2, sst):
        wid = lax.axis_index("s") * NC + lax.axis_index("c")
        base = wid * RPW
        stg = [
            pltpu.async_copy(country_hbm.at[pl.ds(base, RPW)], idc, sst),
            pltpu.async_copy(os_hbm.at[pl.ds(base, RPW)], sto, sst),
            pltpu.async_copy(br_hbm.at[pl.ds(base, RPW)], stb, sst),
            pltpu.async_copy(dv_hbm.at[pl.ds(base, RPW)], std, sst),
        ]
        for h in stg:
            h.wait()
        for i in range(RPW // 16):
            sl = pl.ds(i * 16, 16)
            ids[sl] = sto[sl] * 50 + stb[sl] * 5 + std[sl]
        NB = 3
        buf_a, buf_b = (ba0, ba1, ba2), (bb0, bb1, bb2)
        sem_a, sem_b, sem_o = (sa0, sa1, sa2), (sb0, sb1, sb2), (so0, so1, so2)

        def fire(ch):
            s = ch % NB
            isl = pl.ds(ch * CH, CH)
            return (pltpu.async_copy(pc_hbm.at[idc.at[isl]], buf_a[s], sem_a[s]),
                    pltpu.async_copy(ps_hbm.at[ids.at[isl]], buf_b[s], sem_b[s]))

        gathers = {}
        stores = {}
        for ch in range(min(NB - 1, NCHUNK)):
            gathers[ch] = fire(ch)
        for ch in range(NCHUNK):
            s = ch % NB
            if ch + NB - 1 < NCHUNK:
                if ch >= 1:
                    stores[ch - 1].wait()
                gathers[ch + NB - 1] = fire(ch + NB - 1)
            ga, gb = gathers[ch]
            ga.wait()
            gb.wait()
            a, bb = buf_a[s], buf_b[s]

            def _add_row(r, _, a=a, bb=bb):
                for j in range(D // 16):
                    sl2 = pl.ds(j * 16, 16)
                    plsc.addupdate(a.at[r, sl2], bb[r, sl2])
                return 0

            lax.fori_loop(0, CH, _add_row, 0)
            stores[ch] = pltpu.async_copy(
                a, out_hbm.at[pl.ds(base + ch * CH, CH)], sem_o[s])
        for ch in range(max(0, NCHUNK - NB), NCHUNK):
            stores[ch].wait()

    _SC_ENCODE_CACHE.append(_sc_encode)
    return _sc_encode


def kernel(country, os, browser, device_type, country_table, os_table,
           browser_table, device_type_table, W, b):
    pc, ps = _build_tables(country_table, os_table, browser_table,
                           device_type_table, W, b)
    return _get_sc_encode()(
        pc, ps,
        country.astype(jnp.int32), os.astype(jnp.int32),
        browser.astype(jnp.int32), device_type.astype(jnp.int32))
